# baseline (speedup 1.0000x reference)
import functools
import numpy as np
import jax
import jax.numpy as jnp
from jax import lax
from jax.experimental import pallas as pl
from jax.experimental.pallas import tpu as pltpu
from jax.experimental.pallas import tpu_sc as plsc

W_MAX = 8
STEP = 1
LEAK = 2
KS = (W_MAX - 1) * (STEP + LEAK)
THETA = 512
FODEP = KS
NEURONS = 512
SYNAPSES = 512
BATCH = 32
TIME = 64
TOUT = TIME + KS + 1
TPOT = 96
NV = W_MAX - 1
BN = BATCH * NEURONS
ROW = NEURONS * TOUT


def _base_table():
    t = np.arange(KS, dtype=np.float64)[None, :]
    w = np.arange(W_MAX, dtype=np.float64)[:, None]
    w_step = np.maximum(np.floor(1.0 + t / STEP), 0.0)
    w_leak = np.maximum(np.ceil(w + ((w - 1.0) * STEP - t) / LEAK), 0.0)
    return np.minimum(w_step, w_leak).astype(np.int64)


def _a_cat():
    base = _base_table()
    A = np.zeros((TPOT, NV * TIME), dtype=np.float32)
    for s in range(NV):
        v = s + 1
        for tp in range(TOUT):
            lo = max(0, tp - 1 - (KS - 1))
            hi = min(TIME - 1, tp - 1)
            for t in range(lo, hi + 1):
                A[tp, s * TIME + t] = float(base[v, tp - 1 - t])
    return A


def _pot_kernel(xt_ref, wt_ref, acat_ref, keys_ref, u_ref):
    wt = wt_ref[...]
    xt = xt_ref[...].astype(jnp.bfloat16)
    for s in range(NV):
        m = (wt == (s + 1)).astype(jnp.bfloat16)
        u = jnp.dot(xt, m,
                    preferred_element_type=jnp.float32)
        u_ref[pl.ds(s * TIME, TIME), :, :] = u.reshape(TIME, BATCH, NEURONS)

    u_all = u_ref[...].reshape(NV * TIME, BN)
    pot = jnp.dot(acat_ref[...], u_all,
                  preferred_element_type=jnp.float32)
    pot_i = pot.astype(jnp.int32)

    iota_n = jax.lax.broadcasted_iota(jnp.int32, (TPOT, BN), 1) & (NEURONS - 1)
    key = pot_i * 512 + (NEURONS - 1 - iota_n)
    keys_ref[...] = jnp.max(key.reshape(TPOT, BATCH, NEURONS), axis=2)


def _wta_body(keys_hbm, out_hbm, keys_v, fires_v, wins_v, buf):
    ci = lax.axis_index("c")
    si = lax.axis_index("s")
    kbase = ci * 16
    pltpu.sync_copy(keys_hbm, keys_v)

    zeros16 = jnp.zeros((16,), jnp.int32)

    def zbody(i, carry):
        base = i * 128
        for q in range(8):
            buf[pl.ds(base + q * 16, 16)] = zeros16
        return carry

    lax.fori_loop(0, ROW // 128, zbody, 0)
    for q in range(16 * TPOT // 16):
        fires_v[pl.ds(q * 16, 16)] = zeros16
        wins_v[pl.ds(q * 16, 16)] = zeros16

    iota16 = lax.iota(jnp.int32, 16)
    ones16 = jnp.ones((16,), jnp.int32)
    tkey16 = jnp.full((16,), THETA * 512 + 511, jnp.int32)
    nmask16 = jnp.full((16,), NEURONS - 1, jnp.int32)
    refr16 = jnp.full((16,), FODEP + 1, jnp.int32)
    tpot16 = jnp.full((16,), TPOT, jnp.int32)

    dep_v = zeros16
    for t in range(TOUT):
        row = keys_v[t, pl.ds(kbase, 16)]
        fire_i = (jnp.where(row > tkey16, ones16, zeros16)
                  * jnp.where(dep_v == zeros16, ones16, zeros16))
        win_v = nmask16 - (row & nmask16)
        idx_v = iota16 * tpot16 + jnp.full((16,), t, jnp.int32)
        plsc.store_scatter(fires_v, [idx_v], fire_i)
        plsc.store_scatter(wins_v, [idx_v], win_v)
        dep_v = jnp.maximum(dep_v + fire_i * refr16 - ones16, zeros16)

    mybase = si * TPOT
    tout16 = jnp.full((16,), TOUT, jnp.int32)
    for c2 in range(TPOT // 16):
        f16 = fires_v[pl.ds(mybase + c2 * 16, 16)]
        w16 = wins_v[pl.ds(mybase + c2 * 16, 16)]
        tvec = iota16 + jnp.full((16,), c2 * 16, jnp.int32)
        plsc.store_scatter(buf, [w16 * tout16 + tvec], f16)

    pltpu.sync_copy(buf, out_hbm.at[kbase + si])


def kernel(input_spikes, weight):
    B, C, S, T = input_spikes.shape
    x = input_spikes.reshape(B, C * S, T)
    xt = x.transpose(2, 0, 1).reshape(T * B, C * S).astype(jnp.int8)
    wtT = weight.T.astype(jnp.int32)
    acat = jnp.asarray(_a_cat())

    keys = pl.pallas_call(
        _pot_kernel,
        out_shape=jax.ShapeDtypeStruct((TPOT, BATCH), jnp.int32),
        scratch_shapes=[
            pltpu.VMEM((NV * TIME, BATCH, NEURONS), jnp.float32),
        ],
    )(xt, wtT, acat)

    wta = pl.kernel(
        _wta_body,
        mesh=plsc.VectorSubcoreMesh(core_axis_name="c", subcore_axis_name="s"),
        compiler_params=pltpu.CompilerParams(needs_layout_passes=False),
        out_type=jax.ShapeDtypeStruct((BATCH, ROW), jnp.int32),
        scratch_types=[
            pltpu.VMEM((TPOT, BATCH), jnp.int32),
            pltpu.VMEM((16 * TPOT,), jnp.int32),
            pltpu.VMEM((16 * TPOT,), jnp.int32),
            pltpu.VMEM((ROW,), jnp.int32),
        ],
    )
    out = wta(keys)
    return out.reshape(B, 1, NEURONS, TOUT)


# speedup vs baseline: 1.0007x; 1.0007x over previous
import functools
import numpy as np
import jax
import jax.numpy as jnp
from jax import lax
from jax.experimental import pallas as pl
from jax.experimental.pallas import tpu as pltpu
from jax.experimental.pallas import tpu_sc as plsc

W_MAX = 8
STEP = 1
LEAK = 2
KS = (W_MAX - 1) * (STEP + LEAK)
THETA = 512
FODEP = KS
NEURONS = 512
SYNAPSES = 512
BATCH = 32
TIME = 64
TOUT = TIME + KS + 1
TPOT = 96
NV = W_MAX - 1
BN = BATCH * NEURONS
ROW = NEURONS * TOUT


def _base_table():
    t = np.arange(KS, dtype=np.float64)[None, :]
    w = np.arange(W_MAX, dtype=np.float64)[:, None]
    w_step = np.maximum(np.floor(1.0 + t / STEP), 0.0)
    w_leak = np.maximum(np.ceil(w + ((w - 1.0) * STEP - t) / LEAK), 0.0)
    return np.minimum(w_step, w_leak).astype(np.int64)


def _a_cat():
    base = _base_table()
    A = np.zeros((TPOT, NV * TIME), dtype=np.float32)
    for s in range(NV):
        v = s + 1
        for tp in range(TOUT):
            lo = max(0, tp - 1 - (KS - 1))
            hi = min(TIME - 1, tp - 1)
            for t in range(lo, hi + 1):
                A[tp, s * TIME + t] = float(base[v, tp - 1 - t])
    return A


def _pot_kernel(xt_ref, wt_ref, acat_ref, keys_ref, u_ref):
    wt = wt_ref[...]
    xt = xt_ref[...].astype(jnp.bfloat16)
    for s in range(NV):
        m = (wt == (s + 1)).astype(jnp.bfloat16)
        u = jnp.dot(xt, m,
                    preferred_element_type=jnp.float32)
        u_ref[pl.ds(s * TIME, TIME), :, :] = u.reshape(TIME, BATCH, NEURONS)

    u_all = u_ref[...].reshape(NV * TIME, BN)
    pot = jnp.dot(acat_ref[...], u_all,
                  preferred_element_type=jnp.float32)
    pot_i = pot.astype(jnp.int32)

    iota_n = jax.lax.broadcasted_iota(jnp.int32, (TPOT, BN), 1) & (NEURONS - 1)
    key = pot_i * 512 + (NEURONS - 1 - iota_n)
    keys_ref[...] = jnp.max(key.reshape(TPOT, BATCH, NEURONS), axis=2)


def _wta_body(keys_hbm, out_hbm, keys_v, fires_v, wins_v, buf):
    ci = lax.axis_index("c")
    si = lax.axis_index("s")
    kbase = ci * 16
    pltpu.sync_copy(keys_hbm, keys_v)

    zeros16 = jnp.zeros((16,), jnp.int32)

    def zbody(i, carry):
        base = i * 128
        for q in range(8):
            buf[pl.ds(base + q * 16, 16)] = zeros16
        return carry

    lax.fori_loop(0, ROW // 128, zbody, 0)
    for q in range(16 * TPOT // 16):
        fires_v[pl.ds(q * 16, 16)] = zeros16
        wins_v[pl.ds(q * 16, 16)] = zeros16

    iota16 = lax.iota(jnp.int32, 16)
    ones16 = jnp.ones((16,), jnp.int32)
    tkey16 = jnp.full((16,), THETA * 512 + 511, jnp.int32)
    nmask16 = jnp.full((16,), NEURONS - 1, jnp.int32)
    refr16 = jnp.full((16,), FODEP + 1, jnp.int32)
    tpot16 = jnp.full((16,), TPOT, jnp.int32)

    dep_v = zeros16
    for t in range(TOUT):
        row = keys_v[t, pl.ds(kbase, 16)]
        fire_i = (jnp.where(row > tkey16, ones16, zeros16)
                  * jnp.where(dep_v == zeros16, ones16, zeros16))
        win_v = nmask16 - (row & nmask16)
        idx_v = iota16 * tpot16 + jnp.full((16,), t, jnp.int32)
        plsc.store_scatter(fires_v, [idx_v], fire_i)
        plsc.store_scatter(wins_v, [idx_v], win_v)
        dep_v = jnp.maximum(dep_v + fire_i * refr16 - ones16, zeros16)

    mybase = si * TPOT
    tout16 = jnp.full((16,), TOUT, jnp.int32)
    for c2 in range(TPOT // 16):
        f16 = fires_v[pl.ds(mybase + c2 * 16, 16)]
        w16 = wins_v[pl.ds(mybase + c2 * 16, 16)]
        tvec = iota16 + jnp.full((16,), c2 * 16, jnp.int32)
        plsc.store_scatter(buf, [w16 * tout16 + tvec], f16)

    pltpu.sync_copy(buf, out_hbm.at[kbase + si])


def kernel(input_spikes, weight):
    B, C, S, T = input_spikes.shape
    x8 = input_spikes.reshape(B, C * S, T).astype(jnp.int8)
    xt = x8.transpose(2, 0, 1).reshape(T * B, C * S)
    wtT = weight.T.astype(jnp.int32)
    acat = jnp.asarray(_a_cat())

    keys = pl.pallas_call(
        _pot_kernel,
        out_shape=jax.ShapeDtypeStruct((TPOT, BATCH), jnp.int32),
        scratch_shapes=[
            pltpu.VMEM((NV * TIME, BATCH, NEURONS), jnp.float32),
        ],
    )(xt, wtT, acat)

    wta = pl.kernel(
        _wta_body,
        mesh=plsc.VectorSubcoreMesh(core_axis_name="c", subcore_axis_name="s"),
        compiler_params=pltpu.CompilerParams(needs_layout_passes=False),
        out_type=jax.ShapeDtypeStruct((BATCH, ROW), jnp.int32),
        scratch_types=[
            pltpu.VMEM((TPOT, BATCH), jnp.int32),
            pltpu.VMEM((16 * TPOT,), jnp.int32),
            pltpu.VMEM((16 * TPOT,), jnp.int32),
            pltpu.VMEM((ROW,), jnp.int32),
        ],
    )
    out = wta(keys)
    return out.reshape(B, 1, NEURONS, TOUT)
